# baseline (speedup 1.0000x reference)
import functools

import jax
import jax.numpy as jnp
from jax.experimental import pallas as pl
from jax.experimental.pallas import tpu as pltpu

_S = 21
_P = _S * _S
_STRIDE = 8.0
_OFFSET = 63.0
_HALF = 143.0
_C8 = _STRIDE / _HALF
_EPS = 1e-6
_OUT = 1 + 4 * _P
_CH = 32


def _body(xff_ref, cls_ref, bbox_ref, gx_ref, gy_ref, pxf_ref, pyf_ref,
          out_ref, cls_s, cls_sem):
    f32 = jnp.float32

    cls_cp = pltpu.make_async_copy(cls_ref.at[:, 0, :], cls_s, cls_sem)
    cls_cp.start()

    x1 = bbox_ref[0:1, :]
    y1 = bbox_ref[1:2, :]
    x2 = bbox_ref[2:3, :]
    y2 = bbox_ref[3:4, :]

    ix1 = jnp.clip(((x1 - _OFFSET) / _STRIDE).astype(jnp.int32), 0, _S - 1)
    iy1 = jnp.clip(((y1 - _OFFSET) / _STRIDE).astype(jnp.int32), 0, _S - 1)
    ix2 = jnp.clip(((x2 - _OFFSET) / _STRIDE).astype(jnp.int32), 0, _S - 1)
    iy2 = jnp.clip(((y2 - _OFFSET) / _STRIDE).astype(jnp.int32), 0, _S - 1)
    ix1f = ix1.astype(f32)
    iy1f = iy1.astype(f32)
    ix2f = ix2.astype(f32)
    iy2f = iy2.astype(f32)

    jx1 = ((x1 - _OFFSET) / _STRIDE).astype(jnp.int32)
    jy1 = ((y1 - _OFFSET) / _STRIDE).astype(jnp.int32)
    jx2 = ((x2 - _OFFSET) / _STRIDE).astype(jnp.int32)
    jy2 = ((y2 - _OFFSET) / _STRIDE).astype(jnp.int32)
    w2 = jx2 - jx1
    h2 = jy2 - jy1
    lo_r = jnp.maximum(0, jy1 - h2 // 2)
    hi_r = jnp.minimum(_S, jy2 + 1 + h2 // 2)
    lo_c = jnp.maximum(0, jx1 - w2 // 2)
    hi_c = jnp.minimum(_S, jx2 + 1 + w2 // 2)
    lo_rf = lo_r.astype(f32)
    hi_rf = hi_r.astype(f32)
    lo_cf = lo_c.astype(f32)
    hi_cf = hi_c.astype(f32)

    u_x1 = (_OFFSET - x1) / _HALF
    v_x2 = (x2 - _OFFSET) / _HALF
    u_y1 = (_OFFSET - y1) / _HALF
    v_y2 = (y2 - _OFFSET) / _HALF

    BB = bbox_ref.shape[1]
    cls_acc = jnp.zeros((_CH, BB), f32)
    shape_acc = jnp.zeros((_CH, BB), f32)
    cls_num = jnp.zeros((1, BB), f32)
    shape_num = jnp.zeros((1, BB), f32)

    for p in range(0, _P, _CH):
        n = min(_CH, _P - p)
        sl = slice(p, p + n)
        pxf = pxf_ref[sl, :]
        pyf = pyf_ref[sl, :]
        gx = gx_ref[sl, :]
        gy = gy_ref[sl, :]
        x0 = xff_ref[sl, 0, :]
        x1c = xff_ref[sl, 1, :]
        x2c = xff_ref[sl, 2, :]
        x3c = xff_ref[sl, 3, :]

        t0 = pxf * _C8
        t1 = pyf * _C8
        sad = (jnp.abs(x0 - (t0 + u_x1)) + jnp.abs(x1c - (v_x2 - t0)) +
               jnp.abs(x2c - (t1 + u_y1)) + jnp.abs(x3c - (v_y2 - t1)))
        wxb = ((pyf >= lo_rf) & (pyf < hi_rf) &
               (pxf >= lo_cf) & (pxf < hi_cf))
        shape_t = jnp.where(wxb, sad, 0.0)

        if n == _CH:
            shape_acc = shape_acc + shape_t
        else:
            shape_num = shape_num + jnp.sum(shape_t, axis=0, keepdims=True)

        lo = 1 + 4 * p
        hi = 1 + 4 * (p + n)
        out_ref[lo + 0:hi:4, :] = gx - _HALF * x0
        out_ref[lo + 1:hi:4, :] = gy - _HALF * x2c
        out_ref[lo + 2:hi:4, :] = gx + _HALF * x1c
        out_ref[lo + 3:hi:4, :] = gy + _HALF * x3c

    cls_cp.wait()
    for p in range(0, _P, _CH):
        n = min(_CH, _P - p)
        sl = slice(p, p + n)
        pxf = pxf_ref[sl, :]
        pyf = pyf_ref[sl, :]
        cls = cls_s[sl, :]

        a = pyf - iy1f
        b = iy2f - pyf
        l1 = jnp.minimum(a, b) / (jnp.maximum(a, b) + 1e-4)
        c = pxf - ix1f
        d = ix2f - pxf
        l2 = jnp.minimum(c, d) / (jnp.maximum(c, d) + 1e-4)
        inbox = ((pyf >= iy1f) & (pyf <= iy2f) &
                 (pxf >= ix1f) & (pxf <= ix2f))
        wc33 = jnp.sqrt(l1 * l2)
        cls_t = jnp.where(inbox, jnp.abs(cls - wc33), 0.0)

        if n == _CH:
            cls_acc = cls_acc + cls_t
        else:
            cls_num = cls_num + jnp.sum(cls_t, axis=0, keepdims=True)

    cls_num = cls_num + jnp.sum(cls_acc, axis=0, keepdims=True)
    shape_num = shape_num + jnp.sum(shape_acc, axis=0, keepdims=True)

    cls_cnt = ((iy2 - iy1 + 1) * (ix2 - ix1 + 1)).astype(f32)
    wx_cnt = (jnp.maximum(hi_r - lo_r, 0) *
              jnp.maximum(hi_c - lo_c, 0)).astype(f32)
    out_ref[0:1, :] = (cls_num / (cls_cnt + _EPS) +
                       shape_num / (wx_cnt + _EPS))


@functools.partial(jax.jit)
def kernel(xff, cls3, bbox):
    B = xff.shape[0]
    BB = 128
    xff_t = jnp.transpose(xff, (2, 3, 1, 0)).reshape(_P, 4, B)
    cls_t = jnp.transpose(cls3, (2, 3, 1, 0)).reshape(_P, 1, B)
    bbox_t = jnp.transpose(bbox, (1, 0))

    pix = jnp.arange(_P, dtype=jnp.int32)
    pyf = jnp.broadcast_to((pix // _S)[:, None], (_P, BB)).astype(jnp.float32)
    pxf = jnp.broadcast_to((pix % _S)[:, None], (_P, BB)).astype(jnp.float32)
    gx = _STRIDE * pxf + _OFFSET
    gy = _STRIDE * pyf + _OFFSET

    out_t = pl.pallas_call(
        _body,
        grid=(B // BB,),
        in_specs=[
            pl.BlockSpec((_P, 4, BB), lambda i: (0, 0, i)),
            pl.BlockSpec((_P, 1, BB), lambda i: (0, 0, i)),
            pl.BlockSpec((4, BB), lambda i: (0, i)),
            pl.BlockSpec((_P, BB), lambda i: (0, 0)),
            pl.BlockSpec((_P, BB), lambda i: (0, 0)),
            pl.BlockSpec((_P, BB), lambda i: (0, 0)),
            pl.BlockSpec((_P, BB), lambda i: (0, 0)),
        ],
        out_specs=pl.BlockSpec((_OUT, BB), lambda i: (0, i)),
        out_shape=jax.ShapeDtypeStruct((_OUT, B), jnp.float32),
        scratch_shapes=[pltpu.VMEM((_P, BB), jnp.float32),
                        pltpu.SemaphoreType.DMA],
        compiler_params=pltpu.CompilerParams(
            dimension_semantics=("parallel",),
        ),
    )(xff_t, cls_t, bbox_t, gx, gy, pxf, pyf)
    return jnp.transpose(out_t, (1, 0))


# speedup vs baseline: 1.0008x; 1.0008x over previous
import functools

import jax
import jax.numpy as jnp
from jax.experimental import pallas as pl
from jax.experimental.pallas import tpu as pltpu

_S = 21
_P = _S * _S
_STRIDE = 8.0
_OFFSET = 63.0
_HALF = 143.0
_C8 = _STRIDE / _HALF
_EPS = 1e-6
_OUT = 1 + 4 * _P
_CH = 32


def _body(xff_ref, cls_ref, bbox_ref, gx_ref, gy_ref, pxf_ref, pyf_ref,
          out_ref):
    f32 = jnp.float32

    x1 = bbox_ref[0:1, :]
    y1 = bbox_ref[1:2, :]
    x2 = bbox_ref[2:3, :]
    y2 = bbox_ref[3:4, :]

    ix1 = jnp.clip(((x1 - _OFFSET) / _STRIDE).astype(jnp.int32), 0, _S - 1)
    iy1 = jnp.clip(((y1 - _OFFSET) / _STRIDE).astype(jnp.int32), 0, _S - 1)
    ix2 = jnp.clip(((x2 - _OFFSET) / _STRIDE).astype(jnp.int32), 0, _S - 1)
    iy2 = jnp.clip(((y2 - _OFFSET) / _STRIDE).astype(jnp.int32), 0, _S - 1)
    ix1f = ix1.astype(f32)
    iy1f = iy1.astype(f32)
    ix2f = ix2.astype(f32)
    iy2f = iy2.astype(f32)

    jx1 = ((x1 - _OFFSET) / _STRIDE).astype(jnp.int32)
    jy1 = ((y1 - _OFFSET) / _STRIDE).astype(jnp.int32)
    jx2 = ((x2 - _OFFSET) / _STRIDE).astype(jnp.int32)
    jy2 = ((y2 - _OFFSET) / _STRIDE).astype(jnp.int32)
    w2 = jx2 - jx1
    h2 = jy2 - jy1
    lo_r = jnp.maximum(0, jy1 - h2 // 2)
    hi_r = jnp.minimum(_S, jy2 + 1 + h2 // 2)
    lo_c = jnp.maximum(0, jx1 - w2 // 2)
    hi_c = jnp.minimum(_S, jx2 + 1 + w2 // 2)
    lo_rf = lo_r.astype(f32)
    hi_rf = hi_r.astype(f32)
    lo_cf = lo_c.astype(f32)
    hi_cf = hi_c.astype(f32)

    u_x1 = (_OFFSET - x1) / _HALF
    v_x2 = (x2 - _OFFSET) / _HALF
    u_y1 = (_OFFSET - y1) / _HALF
    v_y2 = (y2 - _OFFSET) / _HALF

    BB = bbox_ref.shape[1]
    cls_acc = jnp.zeros((_CH, BB), f32)
    shape_acc = jnp.zeros((_CH, BB), f32)
    cls_num = jnp.zeros((1, BB), f32)
    shape_num = jnp.zeros((1, BB), f32)

    for p in range(0, _P, _CH):
        n = min(_CH, _P - p)
        sl = slice(p, p + n)
        pxf = pxf_ref[sl, :]
        pyf = pyf_ref[sl, :]
        cls = cls_ref[sl, 0, :]

        a = pyf - iy1f
        b = iy2f - pyf
        l1 = jnp.minimum(a, b) / (jnp.maximum(a, b) + 1e-4)
        c = pxf - ix1f
        d = ix2f - pxf
        l2 = jnp.minimum(c, d) / (jnp.maximum(c, d) + 1e-4)
        inbox = ((pyf >= iy1f) & (pyf <= iy2f) &
                 (pxf >= ix1f) & (pxf <= ix2f))
        wc33 = jnp.sqrt(l1 * l2)
        cls_t = jnp.where(inbox, jnp.abs(cls - wc33), 0.0)

        if n == _CH:
            cls_acc = cls_acc + cls_t
        else:
            cls_num = cls_num + jnp.sum(cls_t, axis=0, keepdims=True)

    for p in range(0, _P, _CH):
        n = min(_CH, _P - p)
        sl = slice(p, p + n)
        pxf = pxf_ref[sl, :]
        pyf = pyf_ref[sl, :]
        gx = gx_ref[sl, :]
        gy = gy_ref[sl, :]
        x0 = xff_ref[sl, 0, :]
        x1c = xff_ref[sl, 1, :]
        x2c = xff_ref[sl, 2, :]
        x3c = xff_ref[sl, 3, :]

        t0 = pxf * _C8
        t1 = pyf * _C8
        sad = (jnp.abs(x0 - (t0 + u_x1)) + jnp.abs(x1c - (v_x2 - t0)) +
               jnp.abs(x2c - (t1 + u_y1)) + jnp.abs(x3c - (v_y2 - t1)))
        wxb = ((pyf >= lo_rf) & (pyf < hi_rf) &
               (pxf >= lo_cf) & (pxf < hi_cf))
        shape_t = jnp.where(wxb, sad, 0.0)

        if n == _CH:
            shape_acc = shape_acc + shape_t
        else:
            shape_num = shape_num + jnp.sum(shape_t, axis=0, keepdims=True)

        lo = 1 + 4 * p
        hi = 1 + 4 * (p + n)
        out_ref[lo + 0:hi:4, :] = gx - _HALF * x0
        out_ref[lo + 1:hi:4, :] = gy - _HALF * x2c
        out_ref[lo + 2:hi:4, :] = gx + _HALF * x1c
        out_ref[lo + 3:hi:4, :] = gy + _HALF * x3c

    cls_num = cls_num + jnp.sum(cls_acc, axis=0, keepdims=True)
    shape_num = shape_num + jnp.sum(shape_acc, axis=0, keepdims=True)

    cls_cnt = ((iy2 - iy1 + 1) * (ix2 - ix1 + 1)).astype(f32)
    wx_cnt = (jnp.maximum(hi_r - lo_r, 0) *
              jnp.maximum(hi_c - lo_c, 0)).astype(f32)
    out_ref[0:1, :] = (cls_num / (cls_cnt + _EPS) +
                       shape_num / (wx_cnt + _EPS))


@functools.partial(jax.jit)
def kernel(xff, cls3, bbox):
    B = xff.shape[0]
    BB = 128
    xff_t = jnp.transpose(xff, (2, 3, 1, 0)).reshape(_P, 4, B)
    cls_t = jnp.transpose(cls3, (2, 3, 1, 0)).reshape(_P, 1, B)
    bbox_t = jnp.transpose(bbox, (1, 0))

    pix = jnp.arange(_P, dtype=jnp.int32)
    pyf = jnp.broadcast_to((pix // _S)[:, None], (_P, BB)).astype(jnp.float32)
    pxf = jnp.broadcast_to((pix % _S)[:, None], (_P, BB)).astype(jnp.float32)
    gx = _STRIDE * pxf + _OFFSET
    gy = _STRIDE * pyf + _OFFSET

    out_t = pl.pallas_call(
        _body,
        grid=(B // BB,),
        in_specs=[
            pl.BlockSpec((_P, 4, BB), lambda i: (0, 0, i)),
            pl.BlockSpec((_P, 1, BB), lambda i: (0, 0, i)),
            pl.BlockSpec((4, BB), lambda i: (0, i)),
            pl.BlockSpec((_P, BB), lambda i: (0, 0)),
            pl.BlockSpec((_P, BB), lambda i: (0, 0)),
            pl.BlockSpec((_P, BB), lambda i: (0, 0)),
            pl.BlockSpec((_P, BB), lambda i: (0, 0)),
        ],
        out_specs=pl.BlockSpec((_OUT, BB), lambda i: (0, i)),
        out_shape=jax.ShapeDtypeStruct((_OUT, B), jnp.float32),
        compiler_params=pltpu.CompilerParams(
            dimension_semantics=("parallel",),
        ),
    )(xff_t, cls_t, bbox_t, gx, gy, pxf, pyf)
    return jnp.transpose(out_t, (1, 0))


# speedup vs baseline: 1.0178x; 1.0170x over previous
import functools

import jax
import jax.numpy as jnp
from jax.experimental import pallas as pl
from jax.experimental.pallas import tpu as pltpu

_S = 21
_P = _S * _S
_STRIDE = 8.0
_OFFSET = 63.0
_HALF = 143.0
_C8 = _STRIDE / _HALF
_EPS = 1e-6
_OUT = 1 + 4 * _P
_CH = 16


def _body(xff_ref, cls_ref, bbox_ref, gx_ref, gy_ref, pxf_ref, pyf_ref,
          out_ref):
    f32 = jnp.float32

    x1 = bbox_ref[0:1, :]
    y1 = bbox_ref[1:2, :]
    x2 = bbox_ref[2:3, :]
    y2 = bbox_ref[3:4, :]

    ix1 = jnp.clip(((x1 - _OFFSET) / _STRIDE).astype(jnp.int32), 0, _S - 1)
    iy1 = jnp.clip(((y1 - _OFFSET) / _STRIDE).astype(jnp.int32), 0, _S - 1)
    ix2 = jnp.clip(((x2 - _OFFSET) / _STRIDE).astype(jnp.int32), 0, _S - 1)
    iy2 = jnp.clip(((y2 - _OFFSET) / _STRIDE).astype(jnp.int32), 0, _S - 1)
    ix1f = ix1.astype(f32)
    iy1f = iy1.astype(f32)
    ix2f = ix2.astype(f32)
    iy2f = iy2.astype(f32)

    jx1 = ((x1 - _OFFSET) / _STRIDE).astype(jnp.int32)
    jy1 = ((y1 - _OFFSET) / _STRIDE).astype(jnp.int32)
    jx2 = ((x2 - _OFFSET) / _STRIDE).astype(jnp.int32)
    jy2 = ((y2 - _OFFSET) / _STRIDE).astype(jnp.int32)
    w2 = jx2 - jx1
    h2 = jy2 - jy1
    lo_r = jnp.maximum(0, jy1 - h2 // 2)
    hi_r = jnp.minimum(_S, jy2 + 1 + h2 // 2)
    lo_c = jnp.maximum(0, jx1 - w2 // 2)
    hi_c = jnp.minimum(_S, jx2 + 1 + w2 // 2)
    lo_rf = lo_r.astype(f32)
    hi_rf = hi_r.astype(f32)
    lo_cf = lo_c.astype(f32)
    hi_cf = hi_c.astype(f32)

    u_x1 = (_OFFSET - x1) / _HALF
    v_x2 = (x2 - _OFFSET) / _HALF
    u_y1 = (_OFFSET - y1) / _HALF
    v_y2 = (y2 - _OFFSET) / _HALF

    BB = bbox_ref.shape[1]
    cls_acc = jnp.zeros((_CH, BB), f32)
    shape_acc = jnp.zeros((_CH, BB), f32)
    cls_num = jnp.zeros((1, BB), f32)
    shape_num = jnp.zeros((1, BB), f32)

    for p in range(0, _P, _CH):
        n = min(_CH, _P - p)
        sl = slice(p, p + n)
        pxf = pxf_ref[sl, :]
        pyf = pyf_ref[sl, :]
        cls = cls_ref[sl, 0, :]

        a = pyf - iy1f
        b = iy2f - pyf
        l1 = jnp.minimum(a, b) / (jnp.maximum(a, b) + 1e-4)
        c = pxf - ix1f
        d = ix2f - pxf
        l2 = jnp.minimum(c, d) / (jnp.maximum(c, d) + 1e-4)
        inbox = ((pyf >= iy1f) & (pyf <= iy2f) &
                 (pxf >= ix1f) & (pxf <= ix2f))
        wc33 = jnp.sqrt(l1 * l2)
        cls_t = jnp.where(inbox, jnp.abs(cls - wc33), 0.0)

        if n == _CH:
            cls_acc = cls_acc + cls_t
        else:
            cls_num = cls_num + jnp.sum(cls_t, axis=0, keepdims=True)

    for p in range(0, _P, _CH):
        n = min(_CH, _P - p)
        sl = slice(p, p + n)
        pxf = pxf_ref[sl, :]
        pyf = pyf_ref[sl, :]
        gx = gx_ref[sl, :]
        gy = gy_ref[sl, :]
        x0 = xff_ref[sl, 0, :]
        x1c = xff_ref[sl, 1, :]
        x2c = xff_ref[sl, 2, :]
        x3c = xff_ref[sl, 3, :]

        t0 = pxf * _C8
        t1 = pyf * _C8
        sad = (jnp.abs(x0 - (t0 + u_x1)) + jnp.abs(x1c - (v_x2 - t0)) +
               jnp.abs(x2c - (t1 + u_y1)) + jnp.abs(x3c - (v_y2 - t1)))
        wxb = ((pyf >= lo_rf) & (pyf < hi_rf) &
               (pxf >= lo_cf) & (pxf < hi_cf))
        shape_t = jnp.where(wxb, sad, 0.0)

        if n == _CH:
            shape_acc = shape_acc + shape_t
        else:
            shape_num = shape_num + jnp.sum(shape_t, axis=0, keepdims=True)

        lo = 1 + 4 * p
        hi = 1 + 4 * (p + n)
        out_ref[lo + 0:hi:4, :] = gx - _HALF * x0
        out_ref[lo + 1:hi:4, :] = gy - _HALF * x2c
        out_ref[lo + 2:hi:4, :] = gx + _HALF * x1c
        out_ref[lo + 3:hi:4, :] = gy + _HALF * x3c

    cls_num = cls_num + jnp.sum(cls_acc, axis=0, keepdims=True)
    shape_num = shape_num + jnp.sum(shape_acc, axis=0, keepdims=True)

    cls_cnt = ((iy2 - iy1 + 1) * (ix2 - ix1 + 1)).astype(f32)
    wx_cnt = (jnp.maximum(hi_r - lo_r, 0) *
              jnp.maximum(hi_c - lo_c, 0)).astype(f32)
    out_ref[0:1, :] = (cls_num / (cls_cnt + _EPS) +
                       shape_num / (wx_cnt + _EPS))


@functools.partial(jax.jit)
def kernel(xff, cls3, bbox):
    B = xff.shape[0]
    BB = 128
    xff_t = jnp.transpose(xff, (2, 3, 1, 0)).reshape(_P, 4, B)
    cls_t = jnp.transpose(cls3, (2, 3, 1, 0)).reshape(_P, 1, B)
    bbox_t = jnp.transpose(bbox, (1, 0))

    pix = jnp.arange(_P, dtype=jnp.int32)
    pyf = jnp.broadcast_to((pix // _S)[:, None], (_P, BB)).astype(jnp.float32)
    pxf = jnp.broadcast_to((pix % _S)[:, None], (_P, BB)).astype(jnp.float32)
    gx = _STRIDE * pxf + _OFFSET
    gy = _STRIDE * pyf + _OFFSET

    out_t = pl.pallas_call(
        _body,
        grid=(B // BB,),
        in_specs=[
            pl.BlockSpec((_P, 4, BB), lambda i: (0, 0, i)),
            pl.BlockSpec((_P, 1, BB), lambda i: (0, 0, i)),
            pl.BlockSpec((4, BB), lambda i: (0, i)),
            pl.BlockSpec((_P, BB), lambda i: (0, 0)),
            pl.BlockSpec((_P, BB), lambda i: (0, 0)),
            pl.BlockSpec((_P, BB), lambda i: (0, 0)),
            pl.BlockSpec((_P, BB), lambda i: (0, 0)),
        ],
        out_specs=pl.BlockSpec((_OUT, BB), lambda i: (0, i)),
        out_shape=jax.ShapeDtypeStruct((_OUT, B), jnp.float32),
        compiler_params=pltpu.CompilerParams(
            dimension_semantics=("parallel",),
        ),
    )(xff_t, cls_t, bbox_t, gx, gy, pxf, pyf)
    return jnp.transpose(out_t, (1, 0))


# speedup vs baseline: 1.0244x; 1.0064x over previous
import functools

import jax
import jax.numpy as jnp
from jax.experimental import pallas as pl
from jax.experimental.pallas import tpu as pltpu

_S = 21
_P = _S * _S
_STRIDE = 8.0
_OFFSET = 63.0
_HALF = 143.0
_C8 = _STRIDE / _HALF
_EPS = 1e-6
_OUT = 1 + 4 * _P
_CH = 8


def _body(xff_ref, cls_ref, bbox_ref, gx_ref, gy_ref, pxf_ref, pyf_ref,
          out_ref):
    f32 = jnp.float32

    x1 = bbox_ref[0:1, :]
    y1 = bbox_ref[1:2, :]
    x2 = bbox_ref[2:3, :]
    y2 = bbox_ref[3:4, :]

    ix1 = jnp.clip(((x1 - _OFFSET) / _STRIDE).astype(jnp.int32), 0, _S - 1)
    iy1 = jnp.clip(((y1 - _OFFSET) / _STRIDE).astype(jnp.int32), 0, _S - 1)
    ix2 = jnp.clip(((x2 - _OFFSET) / _STRIDE).astype(jnp.int32), 0, _S - 1)
    iy2 = jnp.clip(((y2 - _OFFSET) / _STRIDE).astype(jnp.int32), 0, _S - 1)
    ix1f = ix1.astype(f32)
    iy1f = iy1.astype(f32)
    ix2f = ix2.astype(f32)
    iy2f = iy2.astype(f32)

    jx1 = ((x1 - _OFFSET) / _STRIDE).astype(jnp.int32)
    jy1 = ((y1 - _OFFSET) / _STRIDE).astype(jnp.int32)
    jx2 = ((x2 - _OFFSET) / _STRIDE).astype(jnp.int32)
    jy2 = ((y2 - _OFFSET) / _STRIDE).astype(jnp.int32)
    w2 = jx2 - jx1
    h2 = jy2 - jy1
    lo_r = jnp.maximum(0, jy1 - h2 // 2)
    hi_r = jnp.minimum(_S, jy2 + 1 + h2 // 2)
    lo_c = jnp.maximum(0, jx1 - w2 // 2)
    hi_c = jnp.minimum(_S, jx2 + 1 + w2 // 2)
    lo_rf = lo_r.astype(f32)
    hi_rf = hi_r.astype(f32)
    lo_cf = lo_c.astype(f32)
    hi_cf = hi_c.astype(f32)

    u_x1 = (_OFFSET - x1) / _HALF
    v_x2 = (x2 - _OFFSET) / _HALF
    u_y1 = (_OFFSET - y1) / _HALF
    v_y2 = (y2 - _OFFSET) / _HALF

    BB = bbox_ref.shape[1]
    cls_acc = jnp.zeros((_CH, BB), f32)
    shape_acc = jnp.zeros((_CH, BB), f32)
    cls_num = jnp.zeros((1, BB), f32)
    shape_num = jnp.zeros((1, BB), f32)

    for p in range(0, _P, _CH):
        n = min(_CH, _P - p)
        sl = slice(p, p + n)
        pxf = pxf_ref[sl, :]
        pyf = pyf_ref[sl, :]
        cls = cls_ref[sl, 0, :]

        a = pyf - iy1f
        b = iy2f - pyf
        l1 = jnp.minimum(a, b) / (jnp.maximum(a, b) + 1e-4)
        c = pxf - ix1f
        d = ix2f - pxf
        l2 = jnp.minimum(c, d) / (jnp.maximum(c, d) + 1e-4)
        inbox = ((pyf >= iy1f) & (pyf <= iy2f) &
                 (pxf >= ix1f) & (pxf <= ix2f))
        wc33 = jnp.sqrt(l1 * l2)
        cls_t = jnp.where(inbox, jnp.abs(cls - wc33), 0.0)

        if n == _CH:
            cls_acc = cls_acc + cls_t
        else:
            cls_num = cls_num + jnp.sum(cls_t, axis=0, keepdims=True)

    for p in range(0, _P, _CH):
        n = min(_CH, _P - p)
        sl = slice(p, p + n)
        pxf = pxf_ref[sl, :]
        pyf = pyf_ref[sl, :]
        gx = gx_ref[sl, :]
        gy = gy_ref[sl, :]
        x0 = xff_ref[sl, 0, :]
        x1c = xff_ref[sl, 1, :]
        x2c = xff_ref[sl, 2, :]
        x3c = xff_ref[sl, 3, :]

        t0 = pxf * _C8
        t1 = pyf * _C8
        sad = (jnp.abs(x0 - (t0 + u_x1)) + jnp.abs(x1c - (v_x2 - t0)) +
               jnp.abs(x2c - (t1 + u_y1)) + jnp.abs(x3c - (v_y2 - t1)))
        wxb = ((pyf >= lo_rf) & (pyf < hi_rf) &
               (pxf >= lo_cf) & (pxf < hi_cf))
        shape_t = jnp.where(wxb, sad, 0.0)

        if n == _CH:
            shape_acc = shape_acc + shape_t
        else:
            shape_num = shape_num + jnp.sum(shape_t, axis=0, keepdims=True)

        lo = 1 + 4 * p
        hi = 1 + 4 * (p + n)
        out_ref[lo + 0:hi:4, :] = gx - _HALF * x0
        out_ref[lo + 1:hi:4, :] = gy - _HALF * x2c
        out_ref[lo + 2:hi:4, :] = gx + _HALF * x1c
        out_ref[lo + 3:hi:4, :] = gy + _HALF * x3c

    cls_num = cls_num + jnp.sum(cls_acc, axis=0, keepdims=True)
    shape_num = shape_num + jnp.sum(shape_acc, axis=0, keepdims=True)

    cls_cnt = ((iy2 - iy1 + 1) * (ix2 - ix1 + 1)).astype(f32)
    wx_cnt = (jnp.maximum(hi_r - lo_r, 0) *
              jnp.maximum(hi_c - lo_c, 0)).astype(f32)
    out_ref[0:1, :] = (cls_num / (cls_cnt + _EPS) +
                       shape_num / (wx_cnt + _EPS))


@functools.partial(jax.jit)
def kernel(xff, cls3, bbox):
    B = xff.shape[0]
    BB = 128
    xff_t = jnp.transpose(xff, (2, 3, 1, 0)).reshape(_P, 4, B)
    cls_t = jnp.transpose(cls3, (2, 3, 1, 0)).reshape(_P, 1, B)
    bbox_t = jnp.transpose(bbox, (1, 0))

    pix = jnp.arange(_P, dtype=jnp.int32)
    pyf = jnp.broadcast_to((pix // _S)[:, None], (_P, BB)).astype(jnp.float32)
    pxf = jnp.broadcast_to((pix % _S)[:, None], (_P, BB)).astype(jnp.float32)
    gx = _STRIDE * pxf + _OFFSET
    gy = _STRIDE * pyf + _OFFSET

    out_t = pl.pallas_call(
        _body,
        grid=(B // BB,),
        in_specs=[
            pl.BlockSpec((_P, 4, BB), lambda i: (0, 0, i)),
            pl.BlockSpec((_P, 1, BB), lambda i: (0, 0, i)),
            pl.BlockSpec((4, BB), lambda i: (0, i)),
            pl.BlockSpec((_P, BB), lambda i: (0, 0)),
            pl.BlockSpec((_P, BB), lambda i: (0, 0)),
            pl.BlockSpec((_P, BB), lambda i: (0, 0)),
            pl.BlockSpec((_P, BB), lambda i: (0, 0)),
        ],
        out_specs=pl.BlockSpec((_OUT, BB), lambda i: (0, i)),
        out_shape=jax.ShapeDtypeStruct((_OUT, B), jnp.float32),
        compiler_params=pltpu.CompilerParams(
            dimension_semantics=("parallel",),
        ),
    )(xff_t, cls_t, bbox_t, gx, gy, pxf, pyf)
    return jnp.transpose(out_t, (1, 0))
